# baseline (speedup 1.0000x reference)
import functools

import jax
import jax.numpy as jnp
from jax import lax
from jax.experimental import pallas as pl
from jax.experimental.pallas import tpu as pltpu
from jax.experimental.pallas import tpu_sc as plsc

_L = 16
_NS = 16
_C = 96


def _build_sc_kernel(np_t: int, na_e: int, na_r: int, n_chunks: int,
                     half: int):
  assert n_chunks % _NS == 0 and half % _L == 0
  cpt = n_chunks // _NS
  n_slots = -(-(cpt + 1) // 6) * 6
  rows_e = na_e // _NS
  rows_r = na_r // _NS

  mesh = plsc.VectorSubcoreMesh(core_axis_name="c", subcore_axis_name="s")
  f32, i32 = jnp.float32, jnp.int32

  @functools.partial(
      pl.kernel,
      out_type=(
          jax.ShapeDtypeStruct((2 * na_e, half), f32),
          jax.ShapeDtypeStruct((2 * na_r, half), f32),
      ),
      mesh=mesh,
      compiler_params=pltpu.CompilerParams(use_tc_tiling_on_sc=False),
      scratch_types=(
          [pltpu.VMEM_SHARED((na_e + 8, half), f32),
           pltpu.VMEM_SHARED((na_r + 8, half), f32)]
          + [pltpu.VMEM((_C,), i32)] * 9
          + [pltpu.VMEM((_C,), i32)] * 3
          + [pltpu.VMEM((_C, half), f32)] * 6
          + [pltpu.VMEM((_C, half), f32)] * 2
          + [pltpu.SemaphoreType.DMA] * 6
      ),
  )
  def sc_kernel(e2, r2, hh, rr, tt, oe, out_r, acc_e, acc_r,
                ih0, ih1, ih2, ir0, ir1, ir2, it0, it1, it2, ioh, ior, iot,
                gh0, gh1, gr0, gr1, gt0, gt1, vb, mb,
                gsem0, gsem1, isem0, isem1, isem2, ssem):
    c = lax.axis_index("c")
    s = lax.axis_index("s")
    ihs, irs, its = (ih0, ih1, ih2), (ir0, ir1, ir2), (it0, it1, it2)
    ghs, grs, gts = (gh0, gh1), (gr0, gr1), (gt0, gt1)
    gsems, isems = (gsem0, gsem1), (isem0, isem1, isem2)

    coff = c * np_t
    dummy_f = e2.at[pl.ds(0, _C)]
    dummy_i = hh.at[pl.ds(0, _C)]

    pltpu.sync_copy(e2.at[pl.ds(coff + s * rows_e, rows_e)],
                    acc_e.at[pl.ds(s * rows_e, rows_e)])
    pltpu.sync_copy(r2.at[pl.ds(coff + s * rows_r, rows_r)],
                    acc_r.at[pl.ds(s * rows_r, rows_r)])
    plsc.subcore_barrier()

    def fire_idx(g, u):
      base = (g * _NS + s) * _C
      pltpu.async_copy(hh.at[pl.ds(base, _C)], ihs[u], isems[u])
      pltpu.async_copy(rr.at[pl.ds(base, _C)], irs[u], isems[u])
      pltpu.async_copy(tt.at[pl.ds(base, _C)], its[u], isems[u])

    def fire_gathers(u, p):
      for kk in range(_C // _L):
        sl = pl.ds(kk * _L, _L)
        ioh[sl] = ihs[u][sl] + coff
        ior[sl] = irs[u][sl] + coff
        iot[sl] = its[u][sl] + coff
      pltpu.async_copy(e2.at[ioh], ghs[p], gsems[p])
      pltpu.async_copy(r2.at[ior], grs[p], gsems[p])
      pltpu.async_copy(e2.at[iot], gts[p], gsems[p])

    base0 = s * _C
    pltpu.sync_copy(hh.at[pl.ds(base0, _C)], ih0)
    pltpu.sync_copy(rr.at[pl.ds(base0, _C)], ir0)
    pltpu.sync_copy(tt.at[pl.ds(base0, _C)], it0)
    fire_idx(1, 1)
    fire_gathers(0, 0)

    @pl.loop(0, n_slots // 6)
    def _slots(go):
      for b in range(6):
        g = go * 6 + b
        q, w = b % 2, b % 3
        gh, gr, gt = ghs[q], grs[q], gts[q]
        ih, ir, it = ihs[w], irs[w], its[w]

        @pl.when(g < cpt)
        def _(q=q, gh=gh, gr=gr, gt=gt):
          pltpu.make_async_copy(dummy_f, gh, gsems[q]).wait()
          pltpu.make_async_copy(dummy_f, gr, gsems[q]).wait()
          pltpu.make_async_copy(dummy_f, gt, gsems[q]).wait()

        @pl.when(g + 1 < cpt)
        def _(q=q, b=b):
          u = (b + 1) % 3
          pltpu.make_async_copy(dummy_i, ihs[u], isems[u]).wait()
          pltpu.make_async_copy(dummy_i, irs[u], isems[u]).wait()
          pltpu.make_async_copy(dummy_i, its[u], isems[u]).wait()
          fire_gathers(u, 1 - q)

        @pl.when((g >= 1) & (g - 1 < cpt))
        def _():
          pltpu.make_async_copy(dummy_f, vb, ssem).wait()
          pltpu.make_async_copy(dummy_f, vb, ssem).wait()
          pltpu.make_async_copy(dummy_f, vb, ssem).wait()

        @pl.when(g + 2 < cpt)
        def _(g=g, b=b):
          fire_idx(g + 2, (b + 2) % 3)

        @pl.when(g < cpt)
        def _(gh=gh, gr=gr, gt=gt, ih=ih, ir=ir, it=it):
          @pl.loop(0, _C, unroll=4)
          def _rows(row):
            for kk in range(half // _L):
              sl = pl.ds(kk * _L, _L)
              v = gh[row, sl] + gr[row, sl] - gt[row, sl]
              vb[row, sl] = v
              mb[row, sl] = -v

          pltpu.async_copy(mb, acc_e.at[ih], ssem, add=True)
          pltpu.async_copy(vb, acc_e.at[it], ssem, add=True)
          pltpu.async_copy(mb, acc_r.at[ir], ssem, add=True)

    plsc.subcore_barrier()

    def relu_out(acc, out_ref, ocoff, rows):
      done = 0
      while done < rows:
        nb = min(_C, rows - done)
        row0 = s * rows + done
        pltpu.sync_copy(acc.at[pl.ds(row0, nb)], gh0.at[pl.ds(0, nb)])

        @pl.loop(0, nb)
        def _rl(row):
          for kk in range(half // _L):
            sl = pl.ds(kk * _L, _L)
            gh0[row, sl] = jnp.maximum(gh0[row, sl], 0.0)

        pltpu.sync_copy(gh0.at[pl.ds(0, nb)],
                        out_ref.at[pl.ds(ocoff + row0, nb)])
        done += nb

    relu_out(acc_e, oe, c * na_e, rows_e)
    relu_out(acc_r, out_r, c * na_r, rows_r)

  return sc_kernel


def kernel(ent_emb, rel_emb, nei_array):
  n_nodes, d = ent_emb.shape
  n_rels = rel_emb.shape[0]
  n_edges = nei_array.shape[0]
  half = d // 2
  np_t = ((max(n_nodes, n_rels) + 1 + 7) // 8) * 8

  nei = nei_array.astype(jnp.int32)

  blk_edges = _C * _NS
  n_edges_p = -(-n_edges // blk_edges) * blk_edges

  def col(j, dummy_row):
    pad = jnp.full((n_edges_p - n_edges,), dummy_row, jnp.int32)
    return jnp.concatenate([nei[:, j], pad])

  h_idx = col(0, n_nodes)
  r_idx = col(1, n_rels)
  t_idx = col(2, n_nodes)

  def stack(tab):
    n = tab.shape[0]
    pad = jnp.zeros((np_t - n, half), jnp.float32)
    return jnp.concatenate([tab[:, :half], pad, tab[:, half:], pad], axis=0)

  e2 = stack(ent_emb)
  r2 = stack(rel_emb)

  oe2, or2 = _build_sc_kernel(np_t, n_nodes, n_rels, n_edges_p // _C, half)(
      e2, r2, h_idx, r_idx, t_idx)

  ent_out = jnp.concatenate([oe2[:n_nodes], oe2[n_nodes:]], axis=1)
  rel_out = jnp.concatenate([or2[:n_rels], or2[n_rels:]], axis=1)
  return ent_out, rel_out


# speedup vs baseline: 1.6686x; 1.6686x over previous
import functools

import jax
import jax.numpy as jnp
from jax import lax
from jax.experimental import pallas as pl
from jax.experimental.pallas import tpu as pltpu
from jax.experimental.pallas import tpu_sc as plsc

_L = 16
_NS = 16
_C = 96


def _build_sc_kernel(np_t: int, na_e: int, na_r: int, n_chunks: int,
                     half: int):
  assert n_chunks % _NS == 0 and half % _L == 0
  cpt = n_chunks // _NS
  n_slots = -(-(cpt + 1) // 6) * 6
  rows_e = na_e // _NS
  rows_r = na_r // _NS

  mesh = plsc.VectorSubcoreMesh(core_axis_name="c", subcore_axis_name="s")
  f32, i32 = jnp.float32, jnp.int32

  @functools.partial(
      pl.kernel,
      out_type=(
          jax.ShapeDtypeStruct((2 * na_e, half), f32),
          jax.ShapeDtypeStruct((2 * na_r, half), f32),
      ),
      mesh=mesh,
      compiler_params=pltpu.CompilerParams(use_tc_tiling_on_sc=False),
      scratch_types=(
          [pltpu.VMEM_SHARED((na_e + 8, half), f32),
           pltpu.VMEM_SHARED((na_r + 8, half), f32)]
          + [pltpu.VMEM((_C,), i32)] * 9
          + [pltpu.VMEM((_C,), i32)] * 3
          + [pltpu.VMEM((_C, half), f32)] * 6
          + [pltpu.VMEM((_C, half), f32)] * 2
          + [pltpu.SemaphoreType.DMA] * 6
      ),
  )
  def sc_kernel(e2, r2, hh, rr, tt, oe, out_r, acc_e, acc_r,
                ih0, ih1, ih2, ir0, ir1, ir2, it0, it1, it2, ioh, ior, iot,
                gh0, gh1, gr0, gr1, gt0, gt1, vb, mb,
                gsem0, gsem1, isem0, isem1, isem2, ssem):
    c = lax.axis_index("c")
    s = lax.axis_index("s")
    ihs, irs, its = (ih0, ih1, ih2), (ir0, ir1, ir2), (it0, it1, it2)
    ghs, grs, gts = (gh0, gh1), (gr0, gr1), (gt0, gt1)
    gsems, isems = (gsem0, gsem1), (isem0, isem1, isem2)

    coff = c * np_t
    dummy_f = e2.at[pl.ds(0, _C)]
    dummy_i = hh.at[pl.ds(0, _C)]

    pltpu.sync_copy(e2.at[pl.ds(coff + s * rows_e, rows_e)],
                    acc_e.at[pl.ds(s * rows_e, rows_e)])
    pltpu.sync_copy(r2.at[pl.ds(coff + s * rows_r, rows_r)],
                    acc_r.at[pl.ds(s * rows_r, rows_r)])
    plsc.subcore_barrier()

    def fire_idx(g, u):
      base = (g * _NS + s) * _C
      pltpu.async_copy(hh.at[pl.ds(base, _C)], ihs[u], isems[u])
      pltpu.async_copy(rr.at[pl.ds(base, _C)], irs[u], isems[u])
      pltpu.async_copy(tt.at[pl.ds(base, _C)], its[u], isems[u])

    def fire_gathers(u, p):
      for kk in range(_C // _L):
        sl = pl.ds(kk * _L, _L)
        ioh[sl] = ihs[u][sl] + coff
        ior[sl] = irs[u][sl] + coff
        iot[sl] = its[u][sl] + coff
      pltpu.async_copy(e2.at[ioh], ghs[p], gsems[p])
      pltpu.async_copy(r2.at[ior], grs[p], gsems[p])
      pltpu.async_copy(e2.at[iot], gts[p], gsems[p])

    base0 = s * _C
    pltpu.sync_copy(hh.at[pl.ds(base0, _C)], ih0)
    pltpu.sync_copy(rr.at[pl.ds(base0, _C)], ir0)
    pltpu.sync_copy(tt.at[pl.ds(base0, _C)], it0)
    fire_idx(1, 1)
    fire_gathers(0, 0)

    @pl.loop(0, n_slots // 6)
    def _slots(go):
      for b in range(6):
        g = go * 6 + b
        q, w = b % 2, b % 3
        gh, gr, gt = ghs[q], grs[q], gts[q]
        ih, ir, it = ihs[w], irs[w], its[w]

        @pl.when(g < cpt)
        def _(q=q, gh=gh, gr=gr, gt=gt):
          pltpu.make_async_copy(dummy_f, gh, gsems[q]).wait()
          pltpu.make_async_copy(dummy_f, gr, gsems[q]).wait()
          pltpu.make_async_copy(dummy_f, gt, gsems[q]).wait()

        @pl.when(g + 1 < cpt)
        def _(q=q, b=b):
          u = (b + 1) % 3
          pltpu.make_async_copy(dummy_i, ihs[u], isems[u]).wait()
          pltpu.make_async_copy(dummy_i, irs[u], isems[u]).wait()
          pltpu.make_async_copy(dummy_i, its[u], isems[u]).wait()
          fire_gathers(u, 1 - q)

        @pl.when((g >= 1) & (g - 1 < cpt))
        def _():
          pltpu.make_async_copy(dummy_f, vb, ssem).wait()
          pltpu.make_async_copy(dummy_f, vb, ssem).wait()
          pltpu.make_async_copy(dummy_f, vb, ssem).wait()

        @pl.when(g + 2 < cpt)
        def _(g=g, b=b):
          fire_idx(g + 2, (b + 2) % 3)

        @pl.when(g < cpt)
        def _(gh=gh, gr=gr, gt=gt, ih=ih, ir=ir, it=it):
          @pl.loop(0, _C)
          def _rows(row):
            for kk in range(half // _L):
              sl = pl.ds(kk * _L, _L)
              v = gh[row, sl] + gr[row, sl] - gt[row, sl]
              vb[row, sl] = v
              mb[row, sl] = -v

          pltpu.async_copy(mb, acc_e.at[ih], ssem, add=True)
          pltpu.async_copy(vb, acc_e.at[it], ssem, add=True)
          pltpu.async_copy(mb, acc_r.at[ir], ssem, add=True)

    plsc.subcore_barrier()

    def relu_out(acc, out_ref, ocoff, rows):
      done = 0
      while done < rows:
        nb = min(_C, rows - done)
        row0 = s * rows + done
        pltpu.sync_copy(acc.at[pl.ds(row0, nb)], gh0.at[pl.ds(0, nb)])

        @pl.loop(0, nb)
        def _rl(row):
          for kk in range(half // _L):
            sl = pl.ds(kk * _L, _L)
            gh0[row, sl] = jnp.maximum(gh0[row, sl], 0.0)

        pltpu.sync_copy(gh0.at[pl.ds(0, nb)],
                        out_ref.at[pl.ds(ocoff + row0, nb)])
        done += nb

    relu_out(acc_e, oe, c * na_e, rows_e)
    relu_out(acc_r, out_r, c * na_r, rows_r)

  return sc_kernel


def kernel(ent_emb, rel_emb, nei_array):
  n_nodes, d = ent_emb.shape
  n_rels = rel_emb.shape[0]
  n_edges = nei_array.shape[0]
  half = d // 2
  np_t = ((max(n_nodes, n_rels) + 1 + 7) // 8) * 8

  nei = nei_array.astype(jnp.int32)

  blk_edges = _C * _NS
  n_edges_p = -(-n_edges // blk_edges) * blk_edges

  def col(j, dummy_row):
    pad = jnp.full((n_edges_p - n_edges,), dummy_row, jnp.int32)
    return jnp.concatenate([nei[:, j], pad])

  h_idx = col(0, n_nodes)
  r_idx = col(1, n_rels)
  t_idx = col(2, n_nodes)

  def stack(tab):
    n = tab.shape[0]
    pad = jnp.zeros((np_t - n, half), jnp.float32)
    return jnp.concatenate([tab[:, :half], pad, tab[:, half:], pad], axis=0)

  e2 = stack(ent_emb)
  r2 = stack(rel_emb)

  oe2, or2 = _build_sc_kernel(np_t, n_nodes, n_rels, n_edges_p // _C, half)(
      e2, r2, h_idx, r_idx, t_idx)

  ent_out = jnp.concatenate([oe2[:n_nodes], oe2[n_nodes:]], axis=1)
  rel_out = jnp.concatenate([or2[:n_rels], or2[n_rels:]], axis=1)
  return ent_out, rel_out
